# baseline (speedup 1.0000x reference)
import jax
import jax.numpy as jnp
from jax import lax
from jax.experimental import pallas as pl
from jax.experimental.pallas import tpu as pltpu
from jax.experimental.pallas import tpu_sc as plsc

N = 10000
E = 160000
F = 256

NC = 2
NS = 16
NW = NC * NS
RPT = 320
NPAD = NW * RPT
ACC = RPT + 8
DUMP = RPT
CAP = 12288
C = 256
NSC = E // C
K = 64

_mesh = plsc.VectorSubcoreMesh(core_axis_name="c", subcore_axis_name="s")
_sc_params = pltpu.CompilerParams(needs_layout_passes=False)

_I16 = lambda: lax.iota(jnp.int32, 16)


def _sc_part_body(srcH, dstH, bsrc, bloc, counts, sbuf, dbuf, st_s, st_l, cw):
  c = lax.axis_index("c")
  s = lax.axis_index("s")
  wid = c * NS + s
  base = wid * RPT

  def prefill(i, carry):
    st_s[pl.ds(i * 16, 16)] = jnp.zeros((16,), jnp.int32)
    st_l[pl.ds(i * 16, 16)] = jnp.full((16,), DUMP, jnp.int32)
    return carry

  lax.fori_loop(0, CAP // 16, prefill, 0)

  def chunk(j, np_):
    eb = j * C
    pltpu.sync_copy(srcH.at[pl.ds(eb, C)], sbuf)
    pltpu.sync_copy(dstH.at[pl.ds(eb, C)], dbuf)
    for g in range(C // 16):
      d = dbuf[pl.ds(g * 16, 16)]
      sv = sbuf[pl.ds(g * 16, 16)]
      loc = d - base
      m = (loc >= 0) & (loc < RPT)
      m_i = m.astype(jnp.int32)
      cum = plsc.cumsum(m_i)
      pos = np_ + cum - m_i
      plsc.store_scatter(st_s, [pos], sv, mask=m)
      plsc.store_scatter(st_l, [pos], loc, mask=m)
      np_ = jnp.minimum(np_ + cum[15], CAP - 16)
    return np_

  np_ = lax.fori_loop(0, NSC, chunk, jnp.int32(0))

  cw[...] = jnp.full((16,), np_, jnp.int32)
  pltpu.sync_copy(cw, counts.at[pl.ds(wid * 16, 16)])
  pltpu.sync_copy(st_s, bsrc.at[pl.ds(wid * CAP, CAP)])
  pltpu.sync_copy(st_l, bloc.at[pl.ds(wid * CAP, CAP)])


_sc_part = pl.kernel(
    _sc_part_body,
    out_type=(jax.ShapeDtypeStruct((NW * CAP,), jnp.int32),
              jax.ShapeDtypeStruct((NW * CAP,), jnp.int32),
              jax.ShapeDtypeStruct((NW * 16,), jnp.int32)),
    mesh=_mesh,
    compiler_params=_sc_params,
    scratch_types=[
        pltpu.VMEM((C,), jnp.int32),
        pltpu.VMEM((C,), jnp.int32),
        pltpu.VMEM((CAP,), jnp.int32),
        pltpu.VMEM((CAP,), jnp.int32),
        pltpu.VMEM((16,), jnp.int32),
    ])


def _make_sc_layer(first: bool):
  if first:
    out_type = (jax.ShapeDtypeStruct((NPAD, F), jnp.float32),
                jax.ShapeDtypeStruct((NPAD,), jnp.float32))
  else:
    out_type = jax.ShapeDtypeStruct((NPAD, F), jnp.float32)
  scratch = [
      pltpu.VMEM((K,), jnp.int32),
      pltpu.VMEM((K,), jnp.int32),
      pltpu.VMEM((K, F), jnp.float32),
      pltpu.VMEM((ACC, F), jnp.float32),
      pltpu.VMEM((16,), jnp.int32),
  ]
  if first:
    scratch.append(pltpu.VMEM((RPT + 16, ), jnp.float32))

  def body(*refs):
    cols = [_I16() + 16 * k for k in range(F // 16)]
    ones16 = jnp.ones((16,), jnp.float32)
    lane0 = _I16() == 0
    if first:
      (hl, bsrc, bloc, counts, zacc, aggO, invO,
       src_v, loc_v, rows_v, acc, cbuf, cnt_v) = refs
    else:
      (hl, bsrc, bloc, counts, zacc, aggO,
       src_v, loc_v, rows_v, acc, cbuf) = refs
    c = lax.axis_index("c")
    s = lax.axis_index("s")
    wid = c * NS + s

    pltpu.sync_copy(zacc, acc)
    if first:
      def zcnt(i, carry):
        cnt_v[pl.ds(i * 16, 16)] = jnp.zeros((16,), jnp.float32)
        return carry

      lax.fori_loop(0, (RPT + 16) // 16, zcnt, 0)

    pltpu.sync_copy(counts.at[pl.ds(wid * 16, 16)], cbuf)
    cnt_t = cbuf[...][0]
    nch = (cnt_t + (K - 1)) // K

    def chunk(j, carry):
      co = wid * CAP + j * K
      pltpu.sync_copy(bsrc.at[pl.ds(co, K)], src_v)
      pltpu.sync_copy(bloc.at[pl.ds(co, K)], loc_v)
      pltpu.sync_copy(hl.at[src_v], rows_v)

      @plsc.parallel_loop(0, K // 16, unroll=2)
      def egroup(g):
        locs = loc_v[pl.ds(g * 16, 16)]
        for l in range(16):
          loc = locs[l]
          e = g * 16 + l
          for k in range(F // 16):
            r = rows_v[e, pl.ds(k * 16, 16)]
            plsc.addupdate(acc.at[loc, pl.ds(k * 16, 16)], r)
          if first:
            rowv = jnp.full((16,), loc, jnp.int32)
            plsc.addupdate_scatter(cnt_v, [rowv], ones16, mask=lane0)

      return carry

    lax.fori_loop(0, nch, chunk, 0)

    pltpu.sync_copy(acc.at[pl.ds(0, RPT)], aggO.at[pl.ds(wid * RPT, RPT)])
    if first:
      def to_inv(i, carry):
        v = cnt_v[pl.ds(i * 16, 16)]
        cnt_v[pl.ds(i * 16, 16)] = 1.0 / jnp.maximum(v, 1.0)
        return carry

      lax.fori_loop(0, RPT // 16, to_inv, 0)
      pltpu.sync_copy(cnt_v.at[pl.ds(0, RPT)], invO.at[pl.ds(wid * RPT, RPT)])

  return pl.kernel(body, out_type=out_type, mesh=_mesh,
                   compiler_params=_sc_params, scratch_types=scratch)


_sc_layer_first = _make_sc_layer(first=True)
_sc_layer = _make_sc_layer(first=False)


BN = 1000


def _tc_first_body(x_ref, wl_ref, wr_ref, b_ref, hl_ref, hrb_ref):
  h = x_ref[...]
  hl_ref[...] = jnp.dot(h, wl_ref[...], preferred_element_type=jnp.float32)
  hrb_ref[...] = (jnp.dot(h, wr_ref[...], preferred_element_type=jnp.float32)
                  + b_ref[...])


def _tc_mid_body(agg_ref, inv_ref, hrb_ref, wl_ref, wr_ref, b_ref,
                 hl_ref, hrb_out_ref):
  h = jnp.maximum(agg_ref[...] * inv_ref[...] + hrb_ref[...], 0.0)
  hl_ref[...] = jnp.dot(h, wl_ref[...], preferred_element_type=jnp.float32)
  hrb_out_ref[...] = (jnp.dot(h, wr_ref[...],
                              preferred_element_type=jnp.float32) + b_ref[...])


def _tc_last_body(agg_ref, inv_ref, hrb_ref, out_ref):
  out_ref[...] = agg_ref[...] * inv_ref[...] + hrb_ref[...]


_row_spec = pl.BlockSpec((BN, F), lambda i: (i, 0))
_inv_spec = pl.BlockSpec((BN, 1), lambda i: (i, 0))
_w_spec = pl.BlockSpec((F, F), lambda i: (0, 0))
_b_spec = pl.BlockSpec((1, F), lambda i: (0, 0))
_ff_out = (jax.ShapeDtypeStruct((N, F), jnp.float32),
           jax.ShapeDtypeStruct((N, F), jnp.float32))

_tc_first = pl.pallas_call(
    _tc_first_body, grid=(N // BN,),
    in_specs=[_row_spec, _w_spec, _w_spec, _b_spec],
    out_specs=(_row_spec, _row_spec), out_shape=_ff_out)

_tc_mid = pl.pallas_call(
    _tc_mid_body, grid=(N // BN,),
    in_specs=[_row_spec, _inv_spec, _row_spec, _w_spec, _w_spec, _b_spec],
    out_specs=(_row_spec, _row_spec), out_shape=_ff_out)

_tc_last = pl.pallas_call(
    _tc_last_body, grid=(N // BN,),
    in_specs=[_row_spec, _inv_spec, _row_spec],
    out_specs=_row_spec,
    out_shape=jax.ShapeDtypeStruct((N, F), jnp.float32))


def kernel(x, edge_index, Wl1, Wr1, b1, Wl2, Wr2, b2, Wl3, Wr3, b3):
  src = edge_index[0]
  dst = edge_index[1]
  zacc = jnp.zeros((ACC, F), jnp.float32)

  bsrc, bloc, counts = _sc_part(src, dst)
  hl, hrb = _tc_first(x, Wl1, Wr1, b1.reshape(1, F))
  aggp, invp = _sc_layer_first(hl, bsrc, bloc, counts, zacc)
  agg = aggp[:N]
  inv = invp[:N].reshape(N, 1)
  hl, hrb = _tc_mid(agg, inv, hrb, Wl2, Wr2, b2.reshape(1, F))
  agg = _sc_layer(hl, bsrc, bloc, counts, zacc)[:N]
  hl, hrb = _tc_mid(agg, inv, hrb, Wl3, Wr3, b3.reshape(1, F))
  agg = _sc_layer(hl, bsrc, bloc, counts, zacc)[:N]
  return _tc_last(agg, inv, hrb)


# speedup vs baseline: 1.0931x; 1.0931x over previous
import jax
import jax.numpy as jnp
from jax import lax
from jax.experimental import pallas as pl
from jax.experimental.pallas import tpu as pltpu
from jax.experimental.pallas import tpu_sc as plsc

N = 10000
E = 160000
F = 256

NC = 2
NS = 16
NW = NC * NS
RPT = 320
NPAD = NW * RPT
ACC = RPT + 8
DUMP = RPT
CAP = 12288
C = 256
NSC = E // C
K = 64

_mesh = plsc.VectorSubcoreMesh(core_axis_name="c", subcore_axis_name="s")
_sc_params = pltpu.CompilerParams(needs_layout_passes=False)

_I16 = lambda: lax.iota(jnp.int32, 16)


def _sc_part_body(srcH, dstH, bsrc, bloc, counts, sbuf, dbuf, st_s, st_l, cw):
  c = lax.axis_index("c")
  s = lax.axis_index("s")
  wid = c * NS + s
  base = wid * RPT

  def prefill(i, carry):
    st_s[pl.ds(i * 16, 16)] = jnp.zeros((16,), jnp.int32)
    st_l[pl.ds(i * 16, 16)] = jnp.full((16,), DUMP, jnp.int32)
    return carry

  lax.fori_loop(0, CAP // 16, prefill, 0)

  def chunk(j, np_):
    eb = j * C
    pltpu.sync_copy(srcH.at[pl.ds(eb, C)], sbuf)
    pltpu.sync_copy(dstH.at[pl.ds(eb, C)], dbuf)
    for g in range(C // 16):
      d = dbuf[pl.ds(g * 16, 16)]
      sv = sbuf[pl.ds(g * 16, 16)]
      loc = d - base
      m = (loc >= 0) & (loc < RPT)
      m_i = m.astype(jnp.int32)
      cum = plsc.cumsum(m_i)
      pos = np_ + cum - m_i
      plsc.store_scatter(st_s, [pos], sv, mask=m)
      plsc.store_scatter(st_l, [pos], loc, mask=m)
      np_ = jnp.minimum(np_ + cum[15], CAP - 16)
    return np_

  np_ = lax.fori_loop(0, NSC, chunk, jnp.int32(0))

  cw[...] = jnp.full((16,), np_, jnp.int32)
  pltpu.sync_copy(cw, counts.at[pl.ds(wid * 16, 16)])
  pltpu.sync_copy(st_s, bsrc.at[pl.ds(wid * CAP, CAP)])
  pltpu.sync_copy(st_l, bloc.at[pl.ds(wid * CAP, CAP)])


_sc_part = pl.kernel(
    _sc_part_body,
    out_type=(jax.ShapeDtypeStruct((NW * CAP,), jnp.int32),
              jax.ShapeDtypeStruct((NW * CAP,), jnp.int32),
              jax.ShapeDtypeStruct((NW * 16,), jnp.int32)),
    mesh=_mesh,
    compiler_params=_sc_params,
    scratch_types=[
        pltpu.VMEM((C,), jnp.int32),
        pltpu.VMEM((C,), jnp.int32),
        pltpu.VMEM((CAP,), jnp.int32),
        pltpu.VMEM((CAP,), jnp.int32),
        pltpu.VMEM((16,), jnp.int32),
    ])


def _make_sc_layer(first: bool):
  if first:
    out_type = (jax.ShapeDtypeStruct((NPAD, F), jnp.float32),
                jax.ShapeDtypeStruct((NPAD,), jnp.float32))
  else:
    out_type = jax.ShapeDtypeStruct((NPAD, F), jnp.float32)
  scratch = [
      pltpu.VMEM((K,), jnp.int32),
      pltpu.VMEM((K,), jnp.int32),
      pltpu.VMEM((K, F), jnp.float32),
      pltpu.VMEM((ACC, F), jnp.float32),
      pltpu.VMEM((16,), jnp.int32),
  ]
  if first:
    scratch.append(pltpu.VMEM((RPT + 16, ), jnp.float32))

  def body(*refs):
    cols = [_I16() + 16 * k for k in range(F // 16)]
    ones16 = jnp.ones((16,), jnp.float32)
    lane0 = _I16() == 0
    if first:
      (hl, bsrc, bloc, counts, zacc, aggO, invO,
       src_v, loc_v, rows_v, acc, cbuf, cnt_v) = refs
    else:
      (hl, bsrc, bloc, counts, zacc, aggO,
       src_v, loc_v, rows_v, acc, cbuf) = refs
    c = lax.axis_index("c")
    s = lax.axis_index("s")
    wid = c * NS + s

    pltpu.sync_copy(zacc, acc)
    if first:
      def zcnt(i, carry):
        cnt_v[pl.ds(i * 16, 16)] = jnp.zeros((16,), jnp.float32)
        return carry

      lax.fori_loop(0, (RPT + 16) // 16, zcnt, 0)

    pltpu.sync_copy(counts.at[pl.ds(wid * 16, 16)], cbuf)
    cnt_t = cbuf[...][0]
    nch = (cnt_t + (K - 1)) // K

    def chunk(j, carry):
      co = wid * CAP + j * K
      pltpu.sync_copy(bsrc.at[pl.ds(co, K)], src_v)
      pltpu.sync_copy(bloc.at[pl.ds(co, K)], loc_v)
      pltpu.sync_copy(hl.at[src_v], rows_v)

      @plsc.parallel_loop(0, K // 16)
      def egroup(g):
        locs = loc_v[pl.ds(g * 16, 16)]
        for l in range(16):
          loc = locs[l]
          e = g * 16 + l
          for k in range(F // 16):
            r = rows_v[e, pl.ds(k * 16, 16)]
            plsc.addupdate(acc.at[loc, pl.ds(k * 16, 16)], r)
          if first:
            rowv = jnp.full((16,), loc, jnp.int32)
            plsc.addupdate_scatter(cnt_v, [rowv], ones16, mask=lane0)

      return carry

    lax.fori_loop(0, nch, chunk, 0)

    pltpu.sync_copy(acc.at[pl.ds(0, RPT)], aggO.at[pl.ds(wid * RPT, RPT)])
    if first:
      def to_inv(i, carry):
        v = cnt_v[pl.ds(i * 16, 16)]
        cnt_v[pl.ds(i * 16, 16)] = 1.0 / jnp.maximum(v, 1.0)
        return carry

      lax.fori_loop(0, RPT // 16, to_inv, 0)
      pltpu.sync_copy(cnt_v.at[pl.ds(0, RPT)], invO.at[pl.ds(wid * RPT, RPT)])

  return pl.kernel(body, out_type=out_type, mesh=_mesh,
                   compiler_params=_sc_params, scratch_types=scratch)


_sc_layer_first = _make_sc_layer(first=True)
_sc_layer = _make_sc_layer(first=False)


BN = 1000


def _tc_first_body(x_ref, wl_ref, wr_ref, b_ref, hl_ref, hrb_ref):
  h = x_ref[...]
  hl_ref[...] = jnp.dot(h, wl_ref[...], preferred_element_type=jnp.float32)
  hrb_ref[...] = (jnp.dot(h, wr_ref[...], preferred_element_type=jnp.float32)
                  + b_ref[...])


def _tc_mid_body(agg_ref, inv_ref, hrb_ref, wl_ref, wr_ref, b_ref,
                 hl_ref, hrb_out_ref):
  h = jnp.maximum(agg_ref[...] * inv_ref[...] + hrb_ref[...], 0.0)
  hl_ref[...] = jnp.dot(h, wl_ref[...], preferred_element_type=jnp.float32)
  hrb_out_ref[...] = (jnp.dot(h, wr_ref[...],
                              preferred_element_type=jnp.float32) + b_ref[...])


def _tc_last_body(agg_ref, inv_ref, hrb_ref, out_ref):
  out_ref[...] = agg_ref[...] * inv_ref[...] + hrb_ref[...]


_row_spec = pl.BlockSpec((BN, F), lambda i: (i, 0))
_inv_spec = pl.BlockSpec((BN, 1), lambda i: (i, 0))
_w_spec = pl.BlockSpec((F, F), lambda i: (0, 0))
_b_spec = pl.BlockSpec((1, F), lambda i: (0, 0))
_ff_out = (jax.ShapeDtypeStruct((N, F), jnp.float32),
           jax.ShapeDtypeStruct((N, F), jnp.float32))

_tc_first = pl.pallas_call(
    _tc_first_body, grid=(N // BN,),
    in_specs=[_row_spec, _w_spec, _w_spec, _b_spec],
    out_specs=(_row_spec, _row_spec), out_shape=_ff_out)

_tc_mid = pl.pallas_call(
    _tc_mid_body, grid=(N // BN,),
    in_specs=[_row_spec, _inv_spec, _row_spec, _w_spec, _w_spec, _b_spec],
    out_specs=(_row_spec, _row_spec), out_shape=_ff_out)

_tc_last = pl.pallas_call(
    _tc_last_body, grid=(N // BN,),
    in_specs=[_row_spec, _inv_spec, _row_spec],
    out_specs=_row_spec,
    out_shape=jax.ShapeDtypeStruct((N, F), jnp.float32))


def kernel(x, edge_index, Wl1, Wr1, b1, Wl2, Wr2, b2, Wl3, Wr3, b3):
  src = edge_index[0]
  dst = edge_index[1]
  zacc = jnp.zeros((ACC, F), jnp.float32)

  bsrc, bloc, counts = _sc_part(src, dst)
  hl, hrb = _tc_first(x, Wl1, Wr1, b1.reshape(1, F))
  aggp, invp = _sc_layer_first(hl, bsrc, bloc, counts, zacc)
  agg = aggp[:N]
  inv = invp[:N].reshape(N, 1)
  hl, hrb = _tc_mid(agg, inv, hrb, Wl2, Wr2, b2.reshape(1, F))
  agg = _sc_layer(hl, bsrc, bloc, counts, zacc)[:N]
  hl, hrb = _tc_mid(agg, inv, hrb, Wl3, Wr3, b3.reshape(1, F))
  agg = _sc_layer(hl, bsrc, bloc, counts, zacc)[:N]
  return _tc_last(agg, inv, hrb)


# speedup vs baseline: 1.1343x; 1.0378x over previous
import jax
import jax.numpy as jnp
from jax import lax
from jax.experimental import pallas as pl
from jax.experimental.pallas import tpu as pltpu
from jax.experimental.pallas import tpu_sc as plsc

N = 10000
E = 160000
F = 256

NC = 2
NS = 16
NW = NC * NS
RPT = 320
NPAD = NW * RPT
ACC = RPT + 8
DUMP = RPT
CAP = 12288
C = 256
NSC = E // C
K = 64

_mesh = plsc.VectorSubcoreMesh(core_axis_name="c", subcore_axis_name="s")
_sc_params = pltpu.CompilerParams(needs_layout_passes=False)

_I16 = lambda: lax.iota(jnp.int32, 16)


def _sc_part_body(srcH, dstH, bsrc, bloc, counts, sbuf, dbuf, st_s, st_l, cw):
  c = lax.axis_index("c")
  s = lax.axis_index("s")
  wid = c * NS + s
  base = wid * RPT

  def prefill(i, carry):
    st_s[pl.ds(i * 16, 16)] = jnp.zeros((16,), jnp.int32)
    st_l[pl.ds(i * 16, 16)] = jnp.full((16,), DUMP, jnp.int32)
    return carry

  lax.fori_loop(0, CAP // 16, prefill, 0)

  def chunk(j, np0):
    eb = j * C
    pltpu.sync_copy(srcH.at[pl.ds(eb, C)], sbuf)
    pltpu.sync_copy(dstH.at[pl.ds(eb, C)], dbuf)

    @plsc.parallel_loop(0, C // 16, carry=np0)
    def group(g, np_):
      d = dbuf[pl.ds(g * 16, 16)]
      sv = sbuf[pl.ds(g * 16, 16)]
      loc = d - base
      m = (loc >= 0) & (loc < RPT)
      m_i = m.astype(jnp.int32)
      cum = plsc.cumsum(m_i)
      pos = np_ + cum - m_i
      plsc.store_scatter(st_s, [pos], sv, mask=m)
      plsc.store_scatter(st_l, [pos], loc, mask=m)
      return jnp.minimum(np_ + cum[15], CAP - 16)

    return group

  np_ = lax.fori_loop(0, NSC, chunk, jnp.int32(0))

  cw[...] = jnp.full((16,), np_, jnp.int32)
  pltpu.sync_copy(cw, counts.at[pl.ds(wid * 16, 16)])
  pltpu.sync_copy(st_s, bsrc.at[pl.ds(wid * CAP, CAP)])
  pltpu.sync_copy(st_l, bloc.at[pl.ds(wid * CAP, CAP)])


_sc_part = pl.kernel(
    _sc_part_body,
    out_type=(jax.ShapeDtypeStruct((NW * CAP,), jnp.int32),
              jax.ShapeDtypeStruct((NW * CAP,), jnp.int32),
              jax.ShapeDtypeStruct((NW * 16,), jnp.int32)),
    mesh=_mesh,
    compiler_params=_sc_params,
    scratch_types=[
        pltpu.VMEM((C,), jnp.int32),
        pltpu.VMEM((C,), jnp.int32),
        pltpu.VMEM((CAP,), jnp.int32),
        pltpu.VMEM((CAP,), jnp.int32),
        pltpu.VMEM((16,), jnp.int32),
    ])


def _make_sc_layer(first: bool):
  if first:
    out_type = (jax.ShapeDtypeStruct((NPAD, F), jnp.float32),
                jax.ShapeDtypeStruct((NPAD,), jnp.float32))
  else:
    out_type = jax.ShapeDtypeStruct((NPAD, F), jnp.float32)
  scratch = [
      pltpu.VMEM((K,), jnp.int32),
      pltpu.VMEM((K,), jnp.int32),
      pltpu.VMEM((K, F), jnp.float32),
      pltpu.VMEM((ACC, F), jnp.float32),
      pltpu.VMEM((16,), jnp.int32),
  ]
  if first:
    scratch.append(pltpu.VMEM((RPT + 16, ), jnp.float32))

  def body(*refs):
    cols = [_I16() + 16 * k for k in range(F // 16)]
    ones16 = jnp.ones((16,), jnp.float32)
    lane0 = _I16() == 0
    if first:
      (hl, bsrc, bloc, counts, zacc, aggO, invO,
       src_v, loc_v, rows_v, acc, cbuf, cnt_v) = refs
    else:
      (hl, bsrc, bloc, counts, zacc, aggO,
       src_v, loc_v, rows_v, acc, cbuf) = refs
    c = lax.axis_index("c")
    s = lax.axis_index("s")
    wid = c * NS + s

    pltpu.sync_copy(zacc, acc)
    if first:
      def zcnt(i, carry):
        cnt_v[pl.ds(i * 16, 16)] = jnp.zeros((16,), jnp.float32)
        return carry

      lax.fori_loop(0, (RPT + 16) // 16, zcnt, 0)

    pltpu.sync_copy(counts.at[pl.ds(wid * 16, 16)], cbuf)
    cnt_t = cbuf[...][0]
    nch = (cnt_t + (K - 1)) // K

    def chunk(j, carry):
      co = wid * CAP + j * K
      pltpu.sync_copy(bsrc.at[pl.ds(co, K)], src_v)
      pltpu.sync_copy(bloc.at[pl.ds(co, K)], loc_v)
      pltpu.sync_copy(hl.at[src_v], rows_v)

      @plsc.parallel_loop(0, K // 16)
      def egroup(g):
        locs = loc_v[pl.ds(g * 16, 16)]
        for l in range(16):
          loc = locs[l]
          e = g * 16 + l
          for k in range(F // 16):
            r = rows_v[e, pl.ds(k * 16, 16)]
            plsc.addupdate(acc.at[loc, pl.ds(k * 16, 16)], r)
          if first:
            rowv = jnp.full((16,), loc, jnp.int32)
            plsc.addupdate_scatter(cnt_v, [rowv], ones16, mask=lane0)

      return carry

    lax.fori_loop(0, nch, chunk, 0)

    pltpu.sync_copy(acc.at[pl.ds(0, RPT)], aggO.at[pl.ds(wid * RPT, RPT)])
    if first:
      def to_inv(i, carry):
        v = cnt_v[pl.ds(i * 16, 16)]
        cnt_v[pl.ds(i * 16, 16)] = 1.0 / jnp.maximum(v, 1.0)
        return carry

      lax.fori_loop(0, RPT // 16, to_inv, 0)
      pltpu.sync_copy(cnt_v.at[pl.ds(0, RPT)], invO.at[pl.ds(wid * RPT, RPT)])

  return pl.kernel(body, out_type=out_type, mesh=_mesh,
                   compiler_params=_sc_params, scratch_types=scratch)


_sc_layer_first = _make_sc_layer(first=True)
_sc_layer = _make_sc_layer(first=False)


BN = 1000


def _tc_first_body(x_ref, wl_ref, wr_ref, b_ref, hl_ref, hrb_ref):
  h = x_ref[...]
  hl_ref[...] = jnp.dot(h, wl_ref[...], preferred_element_type=jnp.float32)
  hrb_ref[...] = (jnp.dot(h, wr_ref[...], preferred_element_type=jnp.float32)
                  + b_ref[...])


def _tc_mid_body(agg_ref, inv_ref, hrb_ref, wl_ref, wr_ref, b_ref,
                 hl_ref, hrb_out_ref):
  h = jnp.maximum(agg_ref[...] * inv_ref[...] + hrb_ref[...], 0.0)
  hl_ref[...] = jnp.dot(h, wl_ref[...], preferred_element_type=jnp.float32)
  hrb_out_ref[...] = (jnp.dot(h, wr_ref[...],
                              preferred_element_type=jnp.float32) + b_ref[...])


def _tc_last_body(agg_ref, inv_ref, hrb_ref, out_ref):
  out_ref[...] = agg_ref[...] * inv_ref[...] + hrb_ref[...]


_row_spec = pl.BlockSpec((BN, F), lambda i: (i, 0))
_inv_spec = pl.BlockSpec((BN, 1), lambda i: (i, 0))
_w_spec = pl.BlockSpec((F, F), lambda i: (0, 0))
_b_spec = pl.BlockSpec((1, F), lambda i: (0, 0))
_ff_out = (jax.ShapeDtypeStruct((N, F), jnp.float32),
           jax.ShapeDtypeStruct((N, F), jnp.float32))

_tc_first = pl.pallas_call(
    _tc_first_body, grid=(N // BN,),
    in_specs=[_row_spec, _w_spec, _w_spec, _b_spec],
    out_specs=(_row_spec, _row_spec), out_shape=_ff_out)

_tc_mid = pl.pallas_call(
    _tc_mid_body, grid=(N // BN,),
    in_specs=[_row_spec, _inv_spec, _row_spec, _w_spec, _w_spec, _b_spec],
    out_specs=(_row_spec, _row_spec), out_shape=_ff_out)

_tc_last = pl.pallas_call(
    _tc_last_body, grid=(N // BN,),
    in_specs=[_row_spec, _inv_spec, _row_spec],
    out_specs=_row_spec,
    out_shape=jax.ShapeDtypeStruct((N, F), jnp.float32))


def kernel(x, edge_index, Wl1, Wr1, b1, Wl2, Wr2, b2, Wl3, Wr3, b3):
  src = edge_index[0]
  dst = edge_index[1]
  zacc = jnp.zeros((ACC, F), jnp.float32)

  bsrc, bloc, counts = _sc_part(src, dst)
  hl, hrb = _tc_first(x, Wl1, Wr1, b1.reshape(1, F))
  aggp, invp = _sc_layer_first(hl, bsrc, bloc, counts, zacc)
  agg = aggp[:N]
  inv = invp[:N].reshape(N, 1)
  hl, hrb = _tc_mid(agg, inv, hrb, Wl2, Wr2, b2.reshape(1, F))
  agg = _sc_layer(hl, bsrc, bloc, counts, zacc)[:N]
  hl, hrb = _tc_mid(agg, inv, hrb, Wl3, Wr3, b3.reshape(1, F))
  agg = _sc_layer(hl, bsrc, bloc, counts, zacc)[:N]
  return _tc_last(agg, inv, hrb)


# speedup vs baseline: 1.1767x; 1.0373x over previous
import jax
import jax.numpy as jnp
from jax import lax
from jax.experimental import pallas as pl
from jax.experimental.pallas import tpu as pltpu
from jax.experimental.pallas import tpu_sc as plsc

N = 10000
E = 160000
F = 256

NC = 2
NS = 16
NW = NC * NS
RPT = 320
NPAD = NW * RPT
ACC = RPT + 8
DUMP = RPT
CAP = 12288
C = 256
NSC = E // C
K = 64

_mesh = plsc.VectorSubcoreMesh(core_axis_name="c", subcore_axis_name="s")
_sc_params = pltpu.CompilerParams(needs_layout_passes=False)

_I16 = lambda: lax.iota(jnp.int32, 16)


def _sc_part_body(srcH, dstH, bsrc, bloc, counts, sbuf, dbuf, st_s, st_l, cw):
  c = lax.axis_index("c")
  s = lax.axis_index("s")
  wid = c * NS + s
  base = wid * RPT

  def prefill(i, carry):
    st_s[pl.ds(i * 16, 16)] = jnp.zeros((16,), jnp.int32)
    st_l[pl.ds(i * 16, 16)] = jnp.full((16,), DUMP, jnp.int32)
    return carry

  lax.fori_loop(0, CAP // 16, prefill, 0)

  def chunk(j, np0):
    eb = j * C
    pltpu.sync_copy(srcH.at[pl.ds(eb, C)], sbuf)
    pltpu.sync_copy(dstH.at[pl.ds(eb, C)], dbuf)

    @plsc.parallel_loop(0, C // 16, carry=np0)
    def group(g, np_):
      d = dbuf[pl.ds(g * 16, 16)]
      sv = sbuf[pl.ds(g * 16, 16)]
      loc = d - base
      m = (loc >= 0) & (loc < RPT)
      m_i = m.astype(jnp.int32)
      cum = plsc.cumsum(m_i)
      pos = np_ + cum - m_i
      plsc.store_scatter(st_s, [pos], sv, mask=m)
      plsc.store_scatter(st_l, [pos], loc, mask=m)
      return jnp.minimum(np_ + cum[15], CAP - 16)

    return group

  np_ = lax.fori_loop(0, NSC, chunk, jnp.int32(0))

  cw[...] = jnp.full((16,), np_, jnp.int32)
  pltpu.sync_copy(cw, counts.at[pl.ds(wid * 16, 16)])
  pltpu.sync_copy(st_s, bsrc.at[pl.ds(wid * CAP, CAP)])
  pltpu.sync_copy(st_l, bloc.at[pl.ds(wid * CAP, CAP)])


_sc_part = pl.kernel(
    _sc_part_body,
    out_type=(jax.ShapeDtypeStruct((NW * CAP,), jnp.int32),
              jax.ShapeDtypeStruct((NW * CAP,), jnp.int32),
              jax.ShapeDtypeStruct((NW * 16,), jnp.int32)),
    mesh=_mesh,
    compiler_params=_sc_params,
    scratch_types=[
        pltpu.VMEM((C,), jnp.int32),
        pltpu.VMEM((C,), jnp.int32),
        pltpu.VMEM((CAP,), jnp.int32),
        pltpu.VMEM((CAP,), jnp.int32),
        pltpu.VMEM((16,), jnp.int32),
    ])


def _make_sc_layer(first: bool):
  if first:
    out_type = (jax.ShapeDtypeStruct((NPAD, F), jnp.float32),
                jax.ShapeDtypeStruct((NPAD,), jnp.float32))
  else:
    out_type = jax.ShapeDtypeStruct((NPAD, F), jnp.float32)
  scratch = [
      pltpu.VMEM((K,), jnp.int32),
      pltpu.VMEM((K,), jnp.int32),
      pltpu.VMEM((K, F), jnp.float32),
      pltpu.VMEM((K,), jnp.int32),
      pltpu.VMEM((K,), jnp.int32),
      pltpu.VMEM((K, F), jnp.float32),
      pltpu.SemaphoreType.DMA,
      pltpu.SemaphoreType.DMA,
      pltpu.VMEM((ACC, F), jnp.float32),
      pltpu.VMEM((16,), jnp.int32),
  ]
  if first:
    scratch.append(pltpu.VMEM((RPT + 16, ), jnp.float32))

  def body(*refs):
    cols = [_I16() + 16 * k for k in range(F // 16)]
    ones16 = jnp.ones((16,), jnp.float32)
    lane0 = _I16() == 0
    if first:
      (hl, bsrc, bloc, counts, zacc, aggO, invO,
       src_a, loc_a, rows_a, src_b, loc_b, rows_b, sem_a, sem_b,
       acc, cbuf, cnt_v) = refs
    else:
      (hl, bsrc, bloc, counts, zacc, aggO,
       src_a, loc_a, rows_a, src_b, loc_b, rows_b, sem_a, sem_b,
       acc, cbuf) = refs
    c = lax.axis_index("c")
    s = lax.axis_index("s")
    wid = c * NS + s

    pltpu.sync_copy(zacc, acc)
    if first:
      def zcnt(i, carry):
        cnt_v[pl.ds(i * 16, 16)] = jnp.zeros((16,), jnp.float32)
        return carry

      lax.fori_loop(0, (RPT + 16) // 16, zcnt, 0)

    pltpu.sync_copy(counts.at[pl.ds(wid * 16, 16)], cbuf)
    cnt_t = cbuf[...][0]
    nch = (cnt_t + (K - 1)) // K
    nch_e = jnp.maximum(((nch + 1) // 2) * 2, 2)

    def compute(rows_v, loc_v):
      @plsc.parallel_loop(0, K // 16)
      def egroup(g):
        locs = loc_v[pl.ds(g * 16, 16)]
        for l in range(16):
          loc = locs[l]
          e = g * 16 + l
          for k in range(F // 16):
            r = rows_v[e, pl.ds(k * 16, 16)]
            plsc.addupdate(acc.at[loc, pl.ds(k * 16, 16)], r)
          if first:
            rowv = jnp.full((16,), loc, jnp.int32)
            plsc.addupdate_scatter(cnt_v, [rowv], ones16, mask=lane0)

    base = wid * CAP
    pltpu.sync_copy(bsrc.at[pl.ds(base, K)], src_a)
    pltpu.sync_copy(bloc.at[pl.ds(base, K)], loc_a)
    pltpu.make_async_copy(hl.at[src_a], rows_a, sem_a).start()

    def pair(p, carry):
      j0 = 2 * p
      co1 = base + (j0 + 1) * K
      pltpu.sync_copy(bsrc.at[pl.ds(co1, K)], src_b)
      pltpu.sync_copy(bloc.at[pl.ds(co1, K)], loc_b)
      pltpu.make_async_copy(hl.at[src_b], rows_b, sem_b).start()

      pltpu.make_async_copy(hl.at[src_a], rows_a, sem_a).wait()
      compute(rows_a, loc_a)

      @pl.when(j0 + 2 < nch_e)
      def _():
        co2 = base + (j0 + 2) * K
        pltpu.sync_copy(bsrc.at[pl.ds(co2, K)], src_a)
        pltpu.sync_copy(bloc.at[pl.ds(co2, K)], loc_a)
        pltpu.make_async_copy(hl.at[src_a], rows_a, sem_a).start()

      pltpu.make_async_copy(hl.at[src_b], rows_b, sem_b).wait()
      compute(rows_b, loc_b)
      return carry

    lax.fori_loop(0, nch_e // 2, pair, 0)

    pltpu.sync_copy(acc.at[pl.ds(0, RPT)], aggO.at[pl.ds(wid * RPT, RPT)])
    if first:
      def to_inv(i, carry):
        v = cnt_v[pl.ds(i * 16, 16)]
        cnt_v[pl.ds(i * 16, 16)] = 1.0 / jnp.maximum(v, 1.0)
        return carry

      lax.fori_loop(0, RPT // 16, to_inv, 0)
      pltpu.sync_copy(cnt_v.at[pl.ds(0, RPT)], invO.at[pl.ds(wid * RPT, RPT)])

  return pl.kernel(body, out_type=out_type, mesh=_mesh,
                   compiler_params=_sc_params, scratch_types=scratch)


_sc_layer_first = _make_sc_layer(first=True)
_sc_layer = _make_sc_layer(first=False)


BN = 1000


def _tc_first_body(x_ref, wl_ref, wr_ref, b_ref, hl_ref, hrb_ref):
  h = x_ref[...]
  hl_ref[...] = jnp.dot(h, wl_ref[...], preferred_element_type=jnp.float32)
  hrb_ref[...] = (jnp.dot(h, wr_ref[...], preferred_element_type=jnp.float32)
                  + b_ref[...])


def _tc_mid_body(agg_ref, inv_ref, hrb_ref, wl_ref, wr_ref, b_ref,
                 hl_ref, hrb_out_ref):
  h = jnp.maximum(agg_ref[...] * inv_ref[...] + hrb_ref[...], 0.0)
  hl_ref[...] = jnp.dot(h, wl_ref[...], preferred_element_type=jnp.float32)
  hrb_out_ref[...] = (jnp.dot(h, wr_ref[...],
                              preferred_element_type=jnp.float32) + b_ref[...])


def _tc_last_body(agg_ref, inv_ref, hrb_ref, out_ref):
  out_ref[...] = agg_ref[...] * inv_ref[...] + hrb_ref[...]


_row_spec = pl.BlockSpec((BN, F), lambda i: (i, 0))
_inv_spec = pl.BlockSpec((BN, 1), lambda i: (i, 0))
_w_spec = pl.BlockSpec((F, F), lambda i: (0, 0))
_b_spec = pl.BlockSpec((1, F), lambda i: (0, 0))
_ff_out = (jax.ShapeDtypeStruct((N, F), jnp.float32),
           jax.ShapeDtypeStruct((N, F), jnp.float32))

_tc_first = pl.pallas_call(
    _tc_first_body, grid=(N // BN,),
    in_specs=[_row_spec, _w_spec, _w_spec, _b_spec],
    out_specs=(_row_spec, _row_spec), out_shape=_ff_out)

_tc_mid = pl.pallas_call(
    _tc_mid_body, grid=(N // BN,),
    in_specs=[_row_spec, _inv_spec, _row_spec, _w_spec, _w_spec, _b_spec],
    out_specs=(_row_spec, _row_spec), out_shape=_ff_out)

_tc_last = pl.pallas_call(
    _tc_last_body, grid=(N // BN,),
    in_specs=[_row_spec, _inv_spec, _row_spec],
    out_specs=_row_spec,
    out_shape=jax.ShapeDtypeStruct((N, F), jnp.float32))


def kernel(x, edge_index, Wl1, Wr1, b1, Wl2, Wr2, b2, Wl3, Wr3, b3):
  src = edge_index[0]
  dst = edge_index[1]
  zacc = jnp.zeros((ACC, F), jnp.float32)

  bsrc, bloc, counts = _sc_part(src, dst)
  hl, hrb = _tc_first(x, Wl1, Wr1, b1.reshape(1, F))
  aggp, invp = _sc_layer_first(hl, bsrc, bloc, counts, zacc)
  agg = aggp[:N]
  inv = invp[:N].reshape(N, 1)
  hl, hrb = _tc_mid(agg, inv, hrb, Wl2, Wr2, b2.reshape(1, F))
  agg = _sc_layer(hl, bsrc, bloc, counts, zacc)[:N]
  hl, hrb = _tc_mid(agg, inv, hrb, Wl3, Wr3, b3.reshape(1, F))
  agg = _sc_layer(hl, bsrc, bloc, counts, zacc)[:N]
  return _tc_last(agg, inv, hrb)


# speedup vs baseline: 1.5470x; 1.3147x over previous
import jax
import jax.numpy as jnp
from jax import lax
from jax.experimental import pallas as pl
from jax.experimental.pallas import tpu as pltpu
from jax.experimental.pallas import tpu_sc as plsc

N = 10000
E = 160000
F = 256

NC = 2
NS = 16
NW = NC * NS
RPT = 320
NPAD = NW * RPT
ACC = RPT + 8
DUMP = RPT
CAP = 12288
C = 256
NSC = E // C
K = 64

_mesh = plsc.VectorSubcoreMesh(core_axis_name="c", subcore_axis_name="s")
_sc_params = pltpu.CompilerParams(needs_layout_passes=False)

_I16 = lambda: lax.iota(jnp.int32, 16)


def _sc_part_body(srcH, dstH, bsrc, bloc, counts,
                  sb_a, db_a, sb_b, db_b, sem_a, sem_b, st_s, st_l, cw):
  c = lax.axis_index("c")
  s = lax.axis_index("s")
  wid = c * NS + s
  base = wid * RPT

  def prefill(i, carry):
    st_s[pl.ds(i * 16, 16)] = jnp.zeros((16,), jnp.int32)
    st_l[pl.ds(i * 16, 16)] = jnp.full((16,), DUMP, jnp.int32)
    return carry

  lax.fori_loop(0, CAP // 16, prefill, 0)

  def start(j, sbuf, dbuf, sem):
    eb = j * C
    pltpu.make_async_copy(srcH.at[pl.ds(eb, C)], sbuf, sem).start()
    pltpu.make_async_copy(dstH.at[pl.ds(eb, C)], dbuf, sem).start()

  def wait(j, sbuf, dbuf, sem):
    eb = j * C
    pltpu.make_async_copy(srcH.at[pl.ds(eb, C)], sbuf, sem).wait()
    pltpu.make_async_copy(dstH.at[pl.ds(eb, C)], dbuf, sem).wait()

  def scan(sbuf, dbuf, np0):
    @plsc.parallel_loop(0, C // 16, carry=np0)
    def group(g, np_):
      d = dbuf[pl.ds(g * 16, 16)]
      sv = sbuf[pl.ds(g * 16, 16)]
      loc = d - base
      m = (loc >= 0) & (loc < RPT)
      m_i = m.astype(jnp.int32)
      cum = plsc.cumsum(m_i)
      pos = np_ + cum - m_i
      plsc.store_scatter(st_s, [pos], sv, mask=m)
      plsc.store_scatter(st_l, [pos], loc, mask=m)
      return jnp.minimum(np_ + cum[15], CAP - 16)

    return group

  start(0, sb_a, db_a, sem_a)

  def pair(p, np0):
    j0 = 2 * p
    start(j0 + 1, sb_b, db_b, sem_b)
    wait(j0, sb_a, db_a, sem_a)
    np1 = scan(sb_a, db_a, np0)
    start(j0 + 2, sb_a, db_a, sem_a)
    wait(j0 + 1, sb_b, db_b, sem_b)
    return scan(sb_b, db_b, np1)

  np_ = lax.fori_loop(0, NSC // 2, pair, jnp.int32(0))
  wait(NSC - 1, sb_a, db_a, sem_a)
  np_ = scan(sb_a, db_a, np_)

  cw[...] = jnp.full((16,), np_, jnp.int32)
  pltpu.sync_copy(cw, counts.at[pl.ds(wid * 16, 16)])
  pltpu.sync_copy(st_s, bsrc.at[pl.ds(wid * CAP, CAP)])
  pltpu.sync_copy(st_l, bloc.at[pl.ds(wid * CAP, CAP)])


_sc_part = pl.kernel(
    _sc_part_body,
    out_type=(jax.ShapeDtypeStruct((NW * CAP,), jnp.int32),
              jax.ShapeDtypeStruct((NW * CAP,), jnp.int32),
              jax.ShapeDtypeStruct((NW * 16,), jnp.int32)),
    mesh=_mesh,
    compiler_params=_sc_params,
    scratch_types=[
        pltpu.VMEM((C,), jnp.int32),
        pltpu.VMEM((C,), jnp.int32),
        pltpu.VMEM((C,), jnp.int32),
        pltpu.VMEM((C,), jnp.int32),
        pltpu.SemaphoreType.DMA,
        pltpu.SemaphoreType.DMA,
        pltpu.VMEM((CAP,), jnp.int32),
        pltpu.VMEM((CAP,), jnp.int32),
        pltpu.VMEM((16,), jnp.int32),
    ])


def _make_sc_layer(first: bool):
  if first:
    out_type = (jax.ShapeDtypeStruct((NPAD, F), jnp.float32),
                jax.ShapeDtypeStruct((NPAD,), jnp.float32))
  else:
    out_type = jax.ShapeDtypeStruct((NPAD, F), jnp.float32)
  scratch = [
      pltpu.VMEM((K,), jnp.int32),
      pltpu.VMEM((K,), jnp.int32),
      pltpu.VMEM((K, F), jnp.float32),
      pltpu.VMEM((K,), jnp.int32),
      pltpu.VMEM((K,), jnp.int32),
      pltpu.VMEM((K, F), jnp.float32),
      pltpu.SemaphoreType.DMA,
      pltpu.SemaphoreType.DMA,
      pltpu.VMEM((ACC, F), jnp.float32),
      pltpu.VMEM((16,), jnp.int32),
  ]
  if first:
    scratch.append(pltpu.VMEM((RPT + 16, ), jnp.float32))

  def body(*refs):
    cols = [_I16() + 16 * k for k in range(F // 16)]
    ones16 = jnp.ones((16,), jnp.float32)
    lane0 = _I16() == 0
    if first:
      (hl, bsrc, bloc, counts, zacc, aggO, invO,
       src_a, loc_a, rows_a, src_b, loc_b, rows_b, sem_a, sem_b,
       acc, cbuf, cnt_v) = refs
    else:
      (hl, bsrc, bloc, counts, zacc, aggO,
       src_a, loc_a, rows_a, src_b, loc_b, rows_b, sem_a, sem_b,
       acc, cbuf) = refs
    c = lax.axis_index("c")
    s = lax.axis_index("s")
    wid = c * NS + s

    pltpu.sync_copy(zacc, acc)
    if first:
      def zcnt(i, carry):
        cnt_v[pl.ds(i * 16, 16)] = jnp.zeros((16,), jnp.float32)
        return carry

      lax.fori_loop(0, (RPT + 16) // 16, zcnt, 0)

    pltpu.sync_copy(counts.at[pl.ds(wid * 16, 16)], cbuf)
    cnt_t = cbuf[...][0]
    nch = (cnt_t + (K - 1)) // K
    nch_e = jnp.maximum(((nch + 1) // 2) * 2, 2)

    def compute(rows_v, loc_v):
      @plsc.parallel_loop(0, K // 16)
      def egroup(g):
        locs = loc_v[pl.ds(g * 16, 16)]
        for l in range(16):
          loc = locs[l]
          e = g * 16 + l
          for k in range(F // 16):
            r = rows_v[e, pl.ds(k * 16, 16)]
            plsc.addupdate(acc.at[loc, pl.ds(k * 16, 16)], r)
          if first:
            rowv = jnp.full((16,), loc, jnp.int32)
            plsc.addupdate_scatter(cnt_v, [rowv], ones16, mask=lane0)

    base = wid * CAP
    pltpu.sync_copy(bsrc.at[pl.ds(base, K)], src_a)
    pltpu.sync_copy(bloc.at[pl.ds(base, K)], loc_a)
    pltpu.make_async_copy(hl.at[src_a], rows_a, sem_a).start()

    def pair(p, carry):
      j0 = 2 * p
      co1 = base + (j0 + 1) * K
      pltpu.sync_copy(bsrc.at[pl.ds(co1, K)], src_b)
      pltpu.sync_copy(bloc.at[pl.ds(co1, K)], loc_b)
      pltpu.make_async_copy(hl.at[src_b], rows_b, sem_b).start()

      pltpu.make_async_copy(hl.at[src_a], rows_a, sem_a).wait()
      compute(rows_a, loc_a)

      @pl.when(j0 + 2 < nch_e)
      def _():
        co2 = base + (j0 + 2) * K
        pltpu.sync_copy(bsrc.at[pl.ds(co2, K)], src_a)
        pltpu.sync_copy(bloc.at[pl.ds(co2, K)], loc_a)
        pltpu.make_async_copy(hl.at[src_a], rows_a, sem_a).start()

      pltpu.make_async_copy(hl.at[src_b], rows_b, sem_b).wait()
      compute(rows_b, loc_b)
      return carry

    lax.fori_loop(0, nch_e // 2, pair, 0)

    pltpu.sync_copy(acc.at[pl.ds(0, RPT)], aggO.at[pl.ds(wid * RPT, RPT)])
    if first:
      def to_inv(i, carry):
        v = cnt_v[pl.ds(i * 16, 16)]
        cnt_v[pl.ds(i * 16, 16)] = 1.0 / jnp.maximum(v, 1.0)
        return carry

      lax.fori_loop(0, RPT // 16, to_inv, 0)
      pltpu.sync_copy(cnt_v.at[pl.ds(0, RPT)], invO.at[pl.ds(wid * RPT, RPT)])

  return pl.kernel(body, out_type=out_type, mesh=_mesh,
                   compiler_params=_sc_params, scratch_types=scratch)


_sc_layer_first = _make_sc_layer(first=True)
_sc_layer = _make_sc_layer(first=False)


BN = 1000


def _tc_first_body(x_ref, wl_ref, wr_ref, b_ref, hl_ref, hrb_ref):
  h = x_ref[...]
  hl_ref[...] = jnp.dot(h, wl_ref[...], preferred_element_type=jnp.float32)
  hrb_ref[...] = (jnp.dot(h, wr_ref[...], preferred_element_type=jnp.float32)
                  + b_ref[...])


def _tc_mid_body(agg_ref, inv_ref, hrb_ref, wl_ref, wr_ref, b_ref,
                 hl_ref, hrb_out_ref):
  h = jnp.maximum(agg_ref[...] * inv_ref[...] + hrb_ref[...], 0.0)
  hl_ref[...] = jnp.dot(h, wl_ref[...], preferred_element_type=jnp.float32)
  hrb_out_ref[...] = (jnp.dot(h, wr_ref[...],
                              preferred_element_type=jnp.float32) + b_ref[...])


def _tc_last_body(agg_ref, inv_ref, hrb_ref, out_ref):
  out_ref[...] = agg_ref[...] * inv_ref[...] + hrb_ref[...]


_row_spec = pl.BlockSpec((BN, F), lambda i: (i, 0))
_inv_spec = pl.BlockSpec((BN, 1), lambda i: (i, 0))
_w_spec = pl.BlockSpec((F, F), lambda i: (0, 0))
_b_spec = pl.BlockSpec((1, F), lambda i: (0, 0))
_ff_out = (jax.ShapeDtypeStruct((N, F), jnp.float32),
           jax.ShapeDtypeStruct((N, F), jnp.float32))

_tc_first = pl.pallas_call(
    _tc_first_body, grid=(N // BN,),
    in_specs=[_row_spec, _w_spec, _w_spec, _b_spec],
    out_specs=(_row_spec, _row_spec), out_shape=_ff_out)

_tc_mid = pl.pallas_call(
    _tc_mid_body, grid=(N // BN,),
    in_specs=[_row_spec, _inv_spec, _row_spec, _w_spec, _w_spec, _b_spec],
    out_specs=(_row_spec, _row_spec), out_shape=_ff_out)

_tc_last = pl.pallas_call(
    _tc_last_body, grid=(N // BN,),
    in_specs=[_row_spec, _inv_spec, _row_spec],
    out_specs=_row_spec,
    out_shape=jax.ShapeDtypeStruct((N, F), jnp.float32))


def kernel(x, edge_index, Wl1, Wr1, b1, Wl2, Wr2, b2, Wl3, Wr3, b3):
  src = edge_index[0]
  dst = edge_index[1]
  zacc = jnp.zeros((ACC, F), jnp.float32)

  bsrc, bloc, counts = _sc_part(src, dst)
  hl, hrb = _tc_first(x, Wl1, Wr1, b1.reshape(1, F))
  aggp, invp = _sc_layer_first(hl, bsrc, bloc, counts, zacc)
  agg = aggp[:N]
  inv = invp[:N].reshape(N, 1)
  hl, hrb = _tc_mid(agg, inv, hrb, Wl2, Wr2, b2.reshape(1, F))
  agg = _sc_layer(hl, bsrc, bloc, counts, zacc)[:N]
  hl, hrb = _tc_mid(agg, inv, hrb, Wl3, Wr3, b3.reshape(1, F))
  agg = _sc_layer(hl, bsrc, bloc, counts, zacc)[:N]
  return _tc_last(agg, inv, hrb)


# speedup vs baseline: 1.6547x; 1.0696x over previous
import jax
import jax.numpy as jnp
from jax import lax
from jax.experimental import pallas as pl
from jax.experimental.pallas import tpu as pltpu
from jax.experimental.pallas import tpu_sc as plsc

N = 10000
E = 160000
F = 256

NC = 2
NS = 16
NW = NC * NS
RPT = 320
NPAD = NW * RPT
ACC = RPT + 8
DUMP = RPT
CAP = 12288
C = 256
NSC = E // C
K = 64

_mesh = plsc.VectorSubcoreMesh(core_axis_name="c", subcore_axis_name="s")
_sc_params = pltpu.CompilerParams(needs_layout_passes=False)

_I16 = lambda: lax.iota(jnp.int32, 16)


def _sc_part_body(srcH, dstH, bpk, counts,
                  sb_a, db_a, sb_b, db_b, sem_a, sem_b, st_p, cw):
  c = lax.axis_index("c")
  s = lax.axis_index("s")
  wid = c * NS + s
  base = wid * RPT

  def prefill(i, carry):
    st_p[pl.ds(i * 16, 16)] = jnp.full((16,), DUMP, jnp.int32)
    return carry

  lax.fori_loop(0, CAP // 16, prefill, 0)

  def start(j, sbuf, dbuf, sem):
    eb = j * C
    pltpu.make_async_copy(srcH.at[pl.ds(eb, C)], sbuf, sem).start()
    pltpu.make_async_copy(dstH.at[pl.ds(eb, C)], dbuf, sem).start()

  def wait(j, sbuf, dbuf, sem):
    eb = j * C
    pltpu.make_async_copy(srcH.at[pl.ds(eb, C)], sbuf, sem).wait()
    pltpu.make_async_copy(dstH.at[pl.ds(eb, C)], dbuf, sem).wait()

  def scan(sbuf, dbuf, np0):
    @plsc.parallel_loop(0, C // 16, carry=np0)
    def group(g, np_):
      d = dbuf[pl.ds(g * 16, 16)]
      sv = sbuf[pl.ds(g * 16, 16)]
      loc = d - base
      m = (loc >= 0) & (loc < RPT)
      m_i = m.astype(jnp.int32)
      cum = plsc.cumsum(m_i)
      pos = np_ + cum - m_i
      plsc.store_scatter(st_p, [pos], sv * 512 + loc, mask=m)
      return jnp.minimum(np_ + cum[15], CAP - 16)

    return group

  start(0, sb_a, db_a, sem_a)

  def pair(p, np0):
    j0 = 2 * p
    start(j0 + 1, sb_b, db_b, sem_b)
    wait(j0, sb_a, db_a, sem_a)
    np1 = scan(sb_a, db_a, np0)
    start(j0 + 2, sb_a, db_a, sem_a)
    wait(j0 + 1, sb_b, db_b, sem_b)
    return scan(sb_b, db_b, np1)

  np_ = lax.fori_loop(0, NSC // 2, pair, jnp.int32(0))
  wait(NSC - 1, sb_a, db_a, sem_a)
  np_ = scan(sb_a, db_a, np_)

  cw[...] = jnp.full((16,), np_, jnp.int32)
  pltpu.sync_copy(cw, counts.at[pl.ds(wid * 16, 16)])
  pltpu.sync_copy(st_p, bpk.at[pl.ds(wid * CAP, CAP)])


_sc_part = pl.kernel(
    _sc_part_body,
    out_type=(jax.ShapeDtypeStruct((NW * CAP,), jnp.int32),
              jax.ShapeDtypeStruct((NW * 16,), jnp.int32)),
    mesh=_mesh,
    compiler_params=_sc_params,
    scratch_types=[
        pltpu.VMEM((C,), jnp.int32),
        pltpu.VMEM((C,), jnp.int32),
        pltpu.VMEM((C,), jnp.int32),
        pltpu.VMEM((C,), jnp.int32),
        pltpu.SemaphoreType.DMA,
        pltpu.SemaphoreType.DMA,
        pltpu.VMEM((CAP,), jnp.int32),
        pltpu.VMEM((16,), jnp.int32),
    ])


def _make_sc_layer(first: bool):
  if first:
    out_type = (jax.ShapeDtypeStruct((NPAD, F), jnp.float32),
                jax.ShapeDtypeStruct((NPAD,), jnp.float32))
  else:
    out_type = jax.ShapeDtypeStruct((NPAD, F), jnp.float32)
  scratch = [
      pltpu.VMEM((K,), jnp.int32),
      pltpu.VMEM((K,), jnp.int32),
      pltpu.VMEM((K,), jnp.int32),
      pltpu.VMEM((K, F), jnp.float32),
      pltpu.VMEM((K,), jnp.int32),
      pltpu.VMEM((K,), jnp.int32),
      pltpu.VMEM((K,), jnp.int32),
      pltpu.VMEM((K, F), jnp.float32),
      pltpu.SemaphoreType.DMA,
      pltpu.SemaphoreType.DMA,
      pltpu.VMEM((ACC, F), jnp.float32),
      pltpu.VMEM((16,), jnp.int32),
  ]
  if first:
    scratch.append(pltpu.VMEM((RPT + 16, ), jnp.float32))

  def body(*refs):
    ones16 = jnp.ones((16,), jnp.float32)
    lane0 = _I16() == 0
    if first:
      (hl, bpk, counts, zacc, aggO, invO,
       pk_a, src_a, loc_a, rows_a, pk_b, src_b, loc_b, rows_b, sem_a, sem_b,
       acc, cbuf, cnt_v) = refs
    else:
      (hl, bpk, counts, zacc, aggO,
       pk_a, src_a, loc_a, rows_a, pk_b, src_b, loc_b, rows_b, sem_a, sem_b,
       acc, cbuf) = refs
    c = lax.axis_index("c")
    s = lax.axis_index("s")
    wid = c * NS + s

    pltpu.sync_copy(zacc, acc)
    if first:
      def zcnt(i, carry):
        cnt_v[pl.ds(i * 16, 16)] = jnp.zeros((16,), jnp.float32)
        return carry

      lax.fori_loop(0, (RPT + 16) // 16, zcnt, 0)

    pltpu.sync_copy(counts.at[pl.ds(wid * 16, 16)], cbuf)
    cnt_t = cbuf[...][0]
    nch = (cnt_t + (K - 1)) // K
    nch_e = jnp.maximum(((nch + 1) // 2) * 2, 2)

    def compute(rows_v, loc_v):
      @plsc.parallel_loop(0, K // 16)
      def egroup(g):
        locs = loc_v[pl.ds(g * 16, 16)]
        for l in range(16):
          loc = locs[l]
          e = g * 16 + l
          for k in range(F // 16):
            r = rows_v[e, pl.ds(k * 16, 16)]
            plsc.addupdate(acc.at[loc, pl.ds(k * 16, 16)], r)
          if first:
            rowv = jnp.full((16,), loc, jnp.int32)
            plsc.addupdate_scatter(cnt_v, [rowv], ones16, mask=lane0)

    base = wid * CAP

    def load_idx(j, pk_v, src_v, loc_v):
      pltpu.sync_copy(bpk.at[pl.ds(base + j * K, K)], pk_v)
      for g in range(K // 16):
        p16 = pk_v[pl.ds(g * 16, 16)]
        src_v[pl.ds(g * 16, 16)] = lax.shift_right_logical(p16, 9)
        loc_v[pl.ds(g * 16, 16)] = p16 & 511

    load_idx(0, pk_a, src_a, loc_a)
    pltpu.make_async_copy(hl.at[src_a], rows_a, sem_a).start()

    def pair(p, carry):
      j0 = 2 * p
      load_idx(j0 + 1, pk_b, src_b, loc_b)
      pltpu.make_async_copy(hl.at[src_b], rows_b, sem_b).start()

      pltpu.make_async_copy(hl.at[src_a], rows_a, sem_a).wait()
      compute(rows_a, loc_a)

      @pl.when(j0 + 2 < nch_e)
      def _():
        load_idx(j0 + 2, pk_a, src_a, loc_a)
        pltpu.make_async_copy(hl.at[src_a], rows_a, sem_a).start()

      pltpu.make_async_copy(hl.at[src_b], rows_b, sem_b).wait()
      compute(rows_b, loc_b)
      return carry

    lax.fori_loop(0, nch_e // 2, pair, 0)

    pltpu.sync_copy(acc.at[pl.ds(0, RPT)], aggO.at[pl.ds(wid * RPT, RPT)])
    if first:
      def to_inv(i, carry):
        v = cnt_v[pl.ds(i * 16, 16)]
        cnt_v[pl.ds(i * 16, 16)] = 1.0 / jnp.maximum(v, 1.0)
        return carry

      lax.fori_loop(0, RPT // 16, to_inv, 0)
      pltpu.sync_copy(cnt_v.at[pl.ds(0, RPT)], invO.at[pl.ds(wid * RPT, RPT)])

  return pl.kernel(body, out_type=out_type, mesh=_mesh,
                   compiler_params=_sc_params, scratch_types=scratch)


_sc_layer_first = _make_sc_layer(first=True)
_sc_layer = _make_sc_layer(first=False)


BN = 1000


def _tc_first_body(x_ref, wl_ref, wr_ref, b_ref, hl_ref, hrb_ref):
  h = x_ref[...]
  hl_ref[...] = jnp.dot(h, wl_ref[...], preferred_element_type=jnp.float32)
  hrb_ref[...] = (jnp.dot(h, wr_ref[...], preferred_element_type=jnp.float32)
                  + b_ref[...])


def _tc_mid_body(agg_ref, inv_ref, hrb_ref, wl_ref, wr_ref, b_ref,
                 hl_ref, hrb_out_ref):
  h = jnp.maximum(agg_ref[...] * inv_ref[...] + hrb_ref[...], 0.0)
  hl_ref[...] = jnp.dot(h, wl_ref[...], preferred_element_type=jnp.float32)
  hrb_out_ref[...] = (jnp.dot(h, wr_ref[...],
                              preferred_element_type=jnp.float32) + b_ref[...])


def _tc_last_body(agg_ref, inv_ref, hrb_ref, out_ref):
  out_ref[...] = agg_ref[...] * inv_ref[...] + hrb_ref[...]


_row_spec = pl.BlockSpec((BN, F), lambda i: (i, 0))
_inv_spec = pl.BlockSpec((BN, 1), lambda i: (i, 0))
_w_spec = pl.BlockSpec((F, F), lambda i: (0, 0))
_b_spec = pl.BlockSpec((1, F), lambda i: (0, 0))
_ff_out = (jax.ShapeDtypeStruct((N, F), jnp.float32),
           jax.ShapeDtypeStruct((N, F), jnp.float32))

_tc_first = pl.pallas_call(
    _tc_first_body, grid=(N // BN,),
    in_specs=[_row_spec, _w_spec, _w_spec, _b_spec],
    out_specs=(_row_spec, _row_spec), out_shape=_ff_out)

_tc_mid = pl.pallas_call(
    _tc_mid_body, grid=(N // BN,),
    in_specs=[_row_spec, _inv_spec, _row_spec, _w_spec, _w_spec, _b_spec],
    out_specs=(_row_spec, _row_spec), out_shape=_ff_out)

_tc_last = pl.pallas_call(
    _tc_last_body, grid=(N // BN,),
    in_specs=[_row_spec, _inv_spec, _row_spec],
    out_specs=_row_spec,
    out_shape=jax.ShapeDtypeStruct((N, F), jnp.float32))


def kernel(x, edge_index, Wl1, Wr1, b1, Wl2, Wr2, b2, Wl3, Wr3, b3):
  src = edge_index[0]
  dst = edge_index[1]
  zacc = jnp.zeros((ACC, F), jnp.float32)

  bpk, counts = _sc_part(src, dst)
  hl, hrb = _tc_first(x, Wl1, Wr1, b1.reshape(1, F))
  aggp, invp = _sc_layer_first(hl, bpk, counts, zacc)
  agg = aggp[:N]
  inv = invp[:N].reshape(N, 1)
  hl, hrb = _tc_mid(agg, inv, hrb, Wl2, Wr2, b2.reshape(1, F))
  agg = _sc_layer(hl, bpk, counts, zacc)[:N]
  hl, hrb = _tc_mid(agg, inv, hrb, Wl3, Wr3, b3.reshape(1, F))
  agg = _sc_layer(hl, bpk, counts, zacc)[:N]
  return _tc_last(agg, inv, hrb)
